# Initial kernel scaffold; baseline (speedup 1.0000x reference)
import jax
import jax.numpy as jnp
from jax.experimental import pallas as pl


def kernel(idx, token_table, pos_table):
    raise NotImplementedError("write your pallas kernel here")


# speedup vs baseline: 1.2891x; 1.2891x over previous
import functools

import jax
import jax.numpy as jnp
from jax import lax
from jax.experimental import pallas as pl
from jax.experimental.pallas import tpu as pltpu
from jax.experimental.pallas import tpu_sc as plsc

VOCAB = 1000000
EMBED = 64
SEQ = 200
BATCH = 1024

NUM_CORES = 2
NUM_SUBCORES = 16
NUM_WORKERS = NUM_CORES * NUM_SUBCORES
ROWS_PER_WORKER = BATCH // NUM_WORKERS
HALF = SEQ // 2
LANES = 16


def _sc_embed(idx2, token_table, pos_table):
    mesh = plsc.VectorSubcoreMesh(core_axis_name="c", subcore_axis_name="s")

    @functools.partial(
        pl.kernel,
        mesh=mesh,
        out_type=jax.ShapeDtypeStruct((BATCH, SEQ, EMBED), jnp.float32),
        scratch_types=[
            pltpu.VMEM((ROWS_PER_WORKER * 2, HALF), jnp.int32),
            pltpu.VMEM((SEQ, EMBED), jnp.float32),
            pltpu.VMEM((SEQ, EMBED), jnp.float32),
            pltpu.SemaphoreType.DMA,
        ],
        compiler_params=pltpu.CompilerParams(use_tc_tiling_on_sc=False),
    )
    def k(idx_hbm, tok_hbm, pos_hbm, out_hbm, idx_v, pos_v, rows_v, sem):
        wid = lax.axis_index("s") * NUM_CORES + lax.axis_index("c")
        pltpu.sync_copy(pos_hbm, pos_v)
        pltpu.sync_copy(
            idx_hbm.at[pl.ds(wid * (ROWS_PER_WORKER * 2), ROWS_PER_WORKER * 2)],
            idx_v,
        )

        def row_body(r, carry):
            cp1 = pltpu.async_copy(
                tok_hbm.at[idx_v.at[2 * r]], rows_v.at[pl.ds(0, HALF)], sem
            )
            cp2 = pltpu.async_copy(
                tok_hbm.at[idx_v.at[2 * r + 1]], rows_v.at[pl.ds(HALF, HALF)], sem
            )
            cp1.wait()
            cp2.wait()

            def add_body(i, c2):
                for j in range(EMBED // LANES):
                    sl = pl.ds(j * LANES, LANES)
                    plsc.addupdate(rows_v.at[i, sl], pos_v[i, sl])
                return c2

            lax.fori_loop(0, SEQ, add_body, 0)
            pltpu.sync_copy(rows_v, out_hbm.at[wid * ROWS_PER_WORKER + r])
            return carry

        lax.fori_loop(0, ROWS_PER_WORKER, row_body, 0)

    return k(idx2, token_table, pos_table)


def kernel(idx, token_table, pos_table):
    idx2 = idx.astype(jnp.int32).reshape(BATCH * SEQ // HALF, HALF)
    return _sc_embed(idx2, token_table, pos_table)
